# baseline (speedup 1.0000x reference)
import jax
import jax.numpy as jnp
from jax.experimental import pallas as pl
from jax.experimental.pallas import tpu as pltpu

H = 64


def _encoder_body(idx_ref, h0_ref, c0_ref,
                  wih0_ref, whh0_ref, b_ih0_ref, b_hh0_ref,
                  wih1_ref, whh1_ref, b_ih1_ref, b_hh1_ref,
                  table_ref,
                  out_ref, h_ref, c_ref,
                  x_ref, sem):
    i = idx_ref[0]
    cp = pltpu.make_async_copy(table_ref.at[pl.ds(i, 1), :], x_ref, sem)
    cp.start()
    cp.wait()
    x = x_ref[...]

    def cell(xv, hv, cv, wih, whh, b_ih, b_hh):
        dn = (((1,), (1,)), ((), ()))
        gates = (jax.lax.dot_general(xv, wih, dn, preferred_element_type=jnp.float32)
                 + jax.lax.dot_general(hv, whh, dn, preferred_element_type=jnp.float32)
                 + b_ih + b_hh)
        ig = jax.nn.sigmoid(gates[:, 0:H])
        fg = jax.nn.sigmoid(gates[:, H:2 * H])
        gg = jnp.tanh(gates[:, 2 * H:3 * H])
        og = jax.nn.sigmoid(gates[:, 3 * H:4 * H])
        c_new = fg * cv + ig * gg
        h_new = og * jnp.tanh(c_new)
        return h_new, c_new

    h1, c1 = cell(x, h0_ref[0:1, :], c0_ref[0:1, :],
                  wih0_ref[...], whh0_ref[...], b_ih0_ref[...], b_hh0_ref[...])
    h2, c2 = cell(h1, h0_ref[1:2, :], c0_ref[1:2, :],
                  wih1_ref[...], whh1_ref[...], b_ih1_ref[...], b_hh1_ref[...])

    out_ref[...] = h2
    h_ref[0:1, :] = h1
    h_ref[1:2, :] = h2
    c_ref[0:1, :] = c1
    c_ref[1:2, :] = c2


def kernel(input, h0, c0, table, W_ih0, W_hh0, b_ih0, b_hh0, W_ih1, W_hh1, b_ih1, b_hh1):
    f32 = jnp.float32
    out, h_new, c_new = pl.pallas_call(
        _encoder_body,
        in_specs=[
            pl.BlockSpec(memory_space=pltpu.SMEM),
            pl.BlockSpec(memory_space=pltpu.VMEM),
            pl.BlockSpec(memory_space=pltpu.VMEM),
            pl.BlockSpec(memory_space=pltpu.VMEM),
            pl.BlockSpec(memory_space=pltpu.VMEM),
            pl.BlockSpec(memory_space=pltpu.VMEM),
            pl.BlockSpec(memory_space=pltpu.VMEM),
            pl.BlockSpec(memory_space=pltpu.VMEM),
            pl.BlockSpec(memory_space=pltpu.VMEM),
            pl.BlockSpec(memory_space=pltpu.VMEM),
            pl.BlockSpec(memory_space=pltpu.VMEM),
            pl.BlockSpec(memory_space=pl.ANY),
        ],
        out_specs=[
            pl.BlockSpec(memory_space=pltpu.VMEM),
            pl.BlockSpec(memory_space=pltpu.VMEM),
            pl.BlockSpec(memory_space=pltpu.VMEM),
        ],
        out_shape=[
            jax.ShapeDtypeStruct((1, H), f32),
            jax.ShapeDtypeStruct((2, H), f32),
            jax.ShapeDtypeStruct((2, H), f32),
        ],
        scratch_shapes=[
            pltpu.VMEM((1, H), f32),
            pltpu.SemaphoreType.DMA,
        ],
    )(
        input,
        h0.reshape(2, H), c0.reshape(2, H),
        W_ih0, W_hh0, b_ih0.reshape(1, 4 * H), b_hh0.reshape(1, 4 * H),
        W_ih1, W_hh1, b_ih1.reshape(1, 4 * H), b_hh1.reshape(1, 4 * H),
        table,
    )
    return (out.reshape(1, 1, H), h_new.reshape(2, 1, H), c_new.reshape(2, 1, H))
